# Initial kernel scaffold; baseline (speedup 1.0000x reference)
import jax
import jax.numpy as jnp
from jax.experimental import pallas as pl


def kernel(x, edge_index, batch, W_te, b_te, W1, b1, W2, b2):
    raise NotImplementedError("write your pallas kernel here")


# speedup vs baseline: 1.4572x; 1.4572x over previous
import jax
import jax.numpy as jnp
from jax import lax
from jax.experimental import pallas as pl
from jax.experimental.pallas import tpu as pltpu

_N = 50000
_E = 800000
_DIN = 128
_H = 64
_G = 64

_NP = 50048

_R = 1000
_NB = _N // _R


def _split4(q, q_ref):
    q_ref[0] = q[:, 0:16]
    q_ref[1] = q[:, 16:32]
    q_ref[2] = q[:, 32:48]
    q_ref[3] = q[:, 48:64]


def _tc_a_body(x_ref, wte_ref, bte_ref, w1_ref, deg_ref, q_ref):
    h0 = jnp.maximum(
        jnp.dot(x_ref[...], wte_ref[...], preferred_element_type=jnp.float32)
        + bte_ref[...], 0.0)
    p = jnp.dot(h0, w1_ref[...], preferred_element_type=jnp.float32)
    d8 = deg_ref[...]
    dinv = lax.rsqrt(d8[0] + d8[1] + 1.0)[:, 0:1]
    _split4(p * dinv, q_ref)


def _tc_a(x, W_te, b_te2, W1, deg8):
    return pl.pallas_call(
        _tc_a_body,
        grid=(_NB,),
        in_specs=[
            pl.BlockSpec((_R, _DIN), lambda i: (i, 0)),
            pl.BlockSpec((_DIN, _H), lambda i: (0, 0)),
            pl.BlockSpec((1, _H), lambda i: (0, 0)),
            pl.BlockSpec((_H, _H), lambda i: (0, 0)),
            pl.BlockSpec((2, _R, 16), lambda i: (0, i, 0)),
        ],
        out_specs=pl.BlockSpec((4, _R, 16), lambda i: (0, i, 0)),
        out_shape=jax.ShapeDtypeStruct((4, _NP, 16), jnp.float32),
    )(x, W_te, b_te2, W1, deg8)


def _cat_agg(aggA_ref, aggB_ref, q_ref):
    return jnp.concatenate(
        [aggA_ref[0] + q_ref[0], aggA_ref[1] + q_ref[1],
         aggB_ref[0] + q_ref[2], aggB_ref[1] + q_ref[3]], axis=1)


def _tc_b_body(aggA_ref, aggB_ref, q_ref, deg_ref, w2_ref, b1_ref, q2_ref):
    d8 = deg_ref[...]
    dinv = lax.rsqrt(d8[0] + d8[1] + 1.0)[:, 0:1]
    a = _cat_agg(aggA_ref, aggB_ref, q_ref)
    h1 = jnp.maximum(a * dinv + b1_ref[...], 0.0)
    p2 = jnp.dot(h1, w2_ref[...], preferred_element_type=jnp.float32)
    _split4(p2 * dinv, q2_ref)


def _tc_b(aggA, aggB, q1, deg8, W2, b12):
    return pl.pallas_call(
        _tc_b_body,
        grid=(_NB,),
        in_specs=[
            pl.BlockSpec((2, _R, 16), lambda i: (0, i, 0)),
            pl.BlockSpec((2, _R, 16), lambda i: (0, i, 0)),
            pl.BlockSpec((4, _R, 16), lambda i: (0, i, 0)),
            pl.BlockSpec((2, _R, 16), lambda i: (0, i, 0)),
            pl.BlockSpec((_H, _H), lambda i: (0, 0)),
            pl.BlockSpec((1, _H), lambda i: (0, 0)),
        ],
        out_specs=pl.BlockSpec((4, _R, 16), lambda i: (0, i, 0)),
        out_shape=jax.ShapeDtypeStruct((4, _NP, 16), jnp.float32),
    )(aggA, aggB, q1, deg8, W2, b12)


def _tc_c_body(aggA_ref, aggB_ref, q_ref, deg_ref, b2_ref, batch_ref,
               out_ref, sums, cnts):
    i = pl.program_id(0)

    @pl.when(i == 0)
    def _():
        sums[...] = jnp.zeros_like(sums)
        cnts[...] = jnp.zeros_like(cnts)

    d8 = deg_ref[...]
    dinv = lax.rsqrt(d8[0] + d8[1] + 1.0)[:, 0:1]
    a = _cat_agg(aggA_ref, aggB_ref, q_ref)
    h2 = a * dinv + b2_ref[...]
    b = batch_ref[...].reshape(1, _R)
    gids = lax.broadcasted_iota(jnp.int32, (_G, 1), 0)
    onehot = (gids == b).astype(jnp.float32)
    sums[...] += jnp.dot(onehot, h2, preferred_element_type=jnp.float32)
    cnts[...] += jnp.dot(onehot, jnp.ones_like(h2),
                         preferred_element_type=jnp.float32)

    @pl.when(i == pl.num_programs(0) - 1)
    def _():
        out_ref[...] = sums[...] / jnp.maximum(cnts[...], 1.0)


def _tc_c(aggA, aggB, q2, deg8, b22, batch3):
    return pl.pallas_call(
        _tc_c_body,
        grid=(_NB,),
        in_specs=[
            pl.BlockSpec((2, _R, 16), lambda i: (0, i, 0)),
            pl.BlockSpec((2, _R, 16), lambda i: (0, i, 0)),
            pl.BlockSpec((4, _R, 16), lambda i: (0, i, 0)),
            pl.BlockSpec((2, _R, 16), lambda i: (0, i, 0)),
            pl.BlockSpec((1, _H), lambda i: (0, 0)),
            pl.BlockSpec((1, 1, _R), lambda i: (i, 0, 0)),
        ],
        out_specs=pl.BlockSpec((_G, _H), lambda i: (0, 0)),
        out_shape=jax.ShapeDtypeStruct((_G, _H), jnp.float32),
        scratch_shapes=[
            pltpu.VMEM((_G, _H), jnp.float32),
            pltpu.VMEM((_G, _H), jnp.float32),
        ],
    )(aggA, aggB, q2, deg8, b22, batch3)


def kernel(x, edge_index, batch, W_te, b_te, W1, b1, W2, b2):
    src = edge_index[0]
    dst = edge_index[1]
    b_te2 = b_te.reshape(1, _H)
    b12 = b1.reshape(1, _H)
    b22 = b2.reshape(1, _H)
    batch3 = batch.reshape(_NB, 1, _R)

    def _agg(q4, half):
        o = []
        for c in range(2):
            qs = q4[2 * half + c][src]
            o.append(jnp.zeros((_NP, 16), jnp.float32).at[dst].add(qs))
        return jnp.stack(o)

    deg1 = jnp.zeros((_NP,), jnp.float32).at[dst].add(1.0)
    deg8 = jnp.broadcast_to(deg1[None, :, None] * 0.5, (2, _NP, 16))

    q1 = _tc_a(x, W_te, b_te2, W1, deg8)
    agg1A = _agg(q1, 0)
    agg1B = _agg(q1, 1)
    q2 = _tc_b(agg1A, agg1B, q1, deg8, W2, b12)
    agg2A = _agg(q2, 0)
    agg2B = _agg(q2, 1)
    return _tc_c(agg2A, agg2B, q2, deg8, b22, batch3)
